# baseline (speedup 1.0000x reference)
import functools

import jax
import jax.numpy as jnp
from jax import lax
from jax.experimental import pallas as pl
from jax.experimental.pallas import tpu as pltpu
from jax.experimental.pallas import tpu_sc as plsc

NUM_USER = 5000
NUM_ITEM = 5000
N = NUM_USER + NUM_ITEM
E = 160000
EPAD = 163840
D1 = 256
H1 = 128
D2 = 128

NC = 2
NS = 16
L = 16
NW = NC * NS

CEC = 80
CES = 80
ITC = EPAD // CEC // NW
STRIPE = 624
TAIL = N - NS * STRIPE

_mesh = plsc.VectorSubcoreMesh(core_axis_name="c", subcore_axis_name="s")


def _sigmoid(x):
    return 1.0 / (1.0 + jnp.exp(-x))


CCF = 32
ITF = EPAD // CCF // NW


@functools.partial(
    pl.kernel,
    out_type=jax.ShapeDtypeStruct((EPAD,), jnp.float32),
    mesh=_mesh,
    scratch_types=[
        pltpu.VMEM((2, CCF), jnp.int32),
        pltpu.VMEM((2, CCF), jnp.int32),
        pltpu.VMEM((2, CCF, D1), jnp.float32),
        pltpu.VMEM((2, CCF, D1), jnp.float32),
        pltpu.VMEM((CCF, L), jnp.float32),
        pltpu.VMEM((2, CCF), jnp.float32),
        pltpu.SemaphoreType.DMA,
        pltpu.SemaphoreType.DMA,
        pltpu.SemaphoreType.DMA,
        pltpu.SemaphoreType.DMA,
        pltpu.SemaphoreType.DMA,
        pltpu.SemaphoreType.DMA,
    ],
    compiler_params=pltpu.CompilerParams(needs_layout_passes=False),
)
def _edge_conf(xfull, src0, dst0, p_out, srcv, dstv, ra, rc, pacc, pbuf,
               semi0, semi1, semr0, semr1, semp0, semp1):
    semi = (semi0, semi1)
    semr = (semr0, semr1)
    semp = (semp0, semp1)
    c = lax.axis_index("c")
    s = lax.axis_index("s")
    wid = s * NC + c

    def off_of(jj):
        return (wid + jj * NW) * CCF

    def issue_idx(k, jj):
        off = off_of(jj)
        pltpu.async_copy(src0.at[pl.ds(off, CCF)], srcv.at[k], semi[k])
        pltpu.async_copy(dst0.at[pl.ds(off, CCF)], dstv.at[k], semi[k])

    def wait_idx(k, jj):
        off = off_of(jj)
        pltpu.make_async_copy(
            src0.at[pl.ds(off, CCF)], srcv.at[k], semi[k]).wait()
        pltpu.make_async_copy(
            dst0.at[pl.ds(off, CCF)], dstv.at[k], semi[k]).wait()

    def issue_rows(k):
        pltpu.async_copy(xfull.at[srcv.at[k]], ra.at[k], semr[k])
        pltpu.async_copy(xfull.at[dstv.at[k]], rc.at[k], semr[k])

    def wait_rows(k):
        pltpu.make_async_copy(
            xfull.at[srcv.at[k]], ra.at[k], semr[k]).wait()
        pltpu.make_async_copy(
            xfull.at[dstv.at[k]], rc.at[k], semr[k]).wait()

    issue_idx(0, 0)
    wait_idx(0, 0)
    issue_rows(0)
    issue_idx(1, 1)

    def outer(jjj, _):
        for k in (0, 1):
            jj = 2 * jjj + k
            off = off_of(jj)
            wait_rows(k)

            @pl.when(jj + 1 < ITF)
            def _():
                wait_idx(1 - k, jj + 1)
                issue_rows(1 - k)

            @pl.when(jj + 2 < ITF)
            def _():
                issue_idx(k, jj + 2)

            @pl.when(jj >= 2)
            def _():
                poff = off_of(jj - 2)
                pltpu.make_async_copy(
                    pbuf.at[k], p_out.at[pl.ds(poff, CCF)], semp[k]).wait()

            @plsc.parallel_loop(0, CCF, unroll=4)
            def _(i):
                acc = ra[k, i, pl.ds(0, L)] * rc[k, i, pl.ds(0, L)]
                acc2 = ra[k, i, pl.ds(L, L)] * rc[k, i, pl.ds(L, L)]
                for q in range(2, D1 // L, 2):
                    acc = acc + (ra[k, i, pl.ds(q * L, L)]
                                 * rc[k, i, pl.ds(q * L, L)])
                    acc2 = acc2 + (ra[k, i, pl.ds((q + 1) * L, L)]
                                   * rc[k, i, pl.ds((q + 1) * L, L)])
                pacc[i] = acc + acc2
            for g in range(CCF // L):
                lanes = g * L + lax.iota(jnp.int32, L)
                tot = plsc.load_gather(
                    pacc, [lanes, jnp.zeros((L,), jnp.int32)])
                for q in range(1, L):
                    tot = tot + plsc.load_gather(
                        pacc, [lanes, jnp.full((L,), q, jnp.int32)])
                pbuf[k, pl.ds(g * L, L)] = _sigmoid(tot)
            pltpu.async_copy(pbuf.at[k], p_out.at[pl.ds(off, CCF)], semp[k])
        return 0

    lax.fori_loop(0, ITF // 2, outer, 0)
    for k in (0, 1):
        poff = off_of(ITF - 2 + k)
        pltpu.make_async_copy(
            pbuf.at[k], p_out.at[pl.ds(poff, CCF)], semp[k]).wait()


def _make_scatter(H, edge_split):
    scratch = [
        pltpu.VMEM_SHARED((N, H), jnp.float32),
        pltpu.VMEM((2, CES), jnp.int32),
        pltpu.VMEM((2, CES), jnp.int32),
        pltpu.VMEM((2, CES, H), jnp.float32),
        pltpu.VMEM((2, CES, H), jnp.float32),
        pltpu.VMEM((2, CES), jnp.float32),
        pltpu.VMEM((2, CES), jnp.float32),
        pltpu.VMEM((2, CES), jnp.int32),
        pltpu.VMEM((2, CES), jnp.int32),
        pltpu.SemaphoreType.DMA,
        pltpu.SemaphoreType.DMA,
        pltpu.SemaphoreType.DMA,
        pltpu.SemaphoreType.DMA,
        pltpu.SemaphoreType.DMA,
        pltpu.SemaphoreType.DMA,
    ]
    if not edge_split:
        scratch += [
            pltpu.VMEM((2, CES), jnp.int32),
            pltpu.VMEM((2, CES), jnp.int32),
        ]

    if edge_split:
        out_type = jax.ShapeDtypeStruct((2 * N, H), jnp.float32)
    else:
        out_type = [jax.ShapeDtypeStruct((2 * N, H), jnp.float32),
                    jax.ShapeDtypeStruct((N, 2 * H), jnp.float32)]

    @functools.partial(
        pl.kernel,
        out_type=out_type,
        mesh=_mesh,
        scratch_types=scratch,
        compiler_params=pltpu.CompilerParams(needs_layout_passes=False),
    )
    def scatter(xtab, src0, dst0, wf, wb, init, *out_and_scratch):
        if edge_split:
            out = out_and_scratch[0]
            rest = out_and_scratch[1:]
        else:
            out, full_out = out_and_scratch[:2]
            rest = out_and_scratch[2:]
        (acc, srcv, dstv, ra, rc, wfv, wbv, scs, scd,
         semi0, semi1, semr0, semr1, semsc0, semsc1, *offv) = rest
        semi = (semi0, semi1)
        semr = (semr0, semr1)
        semsc = (semsc0, semsc1)
        c = lax.axis_index("c")
        s = lax.axis_index("s")
        coff = c * N
        rbase = s * STRIPE
        pltpu.sync_copy(init.at[pl.ds(coff + rbase, STRIPE)],
                        acc.at[pl.ds(rbase, STRIPE)])

        @pl.when(s == NS - 1)
        def _():
            pltpu.sync_copy(init.at[pl.ds(coff + NS * STRIPE, TAIL)],
                            acc.at[pl.ds(NS * STRIPE, TAIL)])

        if edge_split:
            wid = c * NS + s
            stride = NW
        else:
            wid = s
            stride = NS
        iters = EPAD // CES // stride

        def off_of(jj):
            return (wid + jj * stride) * CES

        def issue_idx(k, jj):
            off = off_of(jj)
            pltpu.async_copy(src0.at[pl.ds(off, CES)], srcv.at[k], semi[k])
            pltpu.async_copy(dst0.at[pl.ds(off, CES)], dstv.at[k], semi[k])
            pltpu.async_copy(wf.at[pl.ds(off, CES)], wfv.at[k], semi[k])
            pltpu.async_copy(wb.at[pl.ds(off, CES)], wbv.at[k], semi[k])

        def wait_idx(k, jj):
            off = off_of(jj)
            pltpu.make_async_copy(
                src0.at[pl.ds(off, CES)], srcv.at[k], semi[k]).wait()
            pltpu.make_async_copy(
                dst0.at[pl.ds(off, CES)], dstv.at[k], semi[k]).wait()
            pltpu.make_async_copy(
                wf.at[pl.ds(off, CES)], wfv.at[k], semi[k]).wait()
            pltpu.make_async_copy(
                wb.at[pl.ds(off, CES)], wbv.at[k], semi[k]).wait()

        def issue_rows(k):
            if edge_split:
                gsrc, gdst = srcv, dstv
            else:
                gsrc, gdst = offv
                for g in range(CES // L):
                    sl = pl.ds(g * L, L)
                    gsrc[k, sl] = srcv[k, sl] + coff
                    gdst[k, sl] = dstv[k, sl] + coff
            pltpu.async_copy(xtab.at[gsrc.at[k]], ra.at[k], semr[k])
            pltpu.async_copy(xtab.at[gdst.at[k]], rc.at[k], semr[k])

        def wait_rows(k):
            gsrc, gdst = (srcv, dstv) if edge_split else offv
            pltpu.make_async_copy(
                xtab.at[gsrc.at[k]], ra.at[k], semr[k]).wait()
            pltpu.make_async_copy(
                xtab.at[gdst.at[k]], rc.at[k], semr[k]).wait()

        def wait_scatter(k):
            pltpu.make_async_copy(
                ra.at[k], acc.at[scd.at[k]], semsc[k]).wait()
            pltpu.make_async_copy(
                rc.at[k], acc.at[scs.at[k]], semsc[k]).wait()

        issue_idx(0, 0)
        wait_idx(0, 0)
        issue_rows(0)
        issue_idx(1, 1)
        plsc.subcore_barrier()

        def outer(jjj, _):
            for k in (0, 1):
                jj = 2 * jjj + k
                wait_rows(k)

                @pl.when(jj + 1 < iters)
                def _():
                    wait_idx(1 - k, jj + 1)

                    @pl.when(jj >= 1)
                    def _():
                        wait_scatter(1 - k)

                    issue_rows(1 - k)

                for g in range(CES // L):
                    sl = pl.ds(g * L, L)
                    scs[k, sl] = srcv[k, sl]
                    scd[k, sl] = dstv[k, sl]

                @plsc.parallel_loop(0, CES, unroll=4)
                def _(i):
                    wfb = plsc.load_gather(
                        wfv.at[k], [jnp.full((L,), i, jnp.int32)])
                    wbb = plsc.load_gather(
                        wbv.at[k], [jnp.full((L,), i, jnp.int32)])
                    for q in range(H // L):
                        sl = pl.ds(q * L, L)
                        ra[k, i, sl] = ra[k, i, sl] * wfb
                        rc[k, i, sl] = rc[k, i, sl] * wbb
                pltpu.async_copy(ra.at[k], acc.at[scd.at[k]], semsc[k],
                                 add=True)
                pltpu.async_copy(rc.at[k], acc.at[scs.at[k]], semsc[k],
                                 add=True)

                @pl.when(jj + 2 < iters)
                def _():
                    issue_idx(k, jj + 2)
            return 0

        lax.fori_loop(0, iters // 2, outer, 0)
        for k in (0, 1):
            pltpu.make_async_copy(
                ra.at[k], acc.at[scd.at[k]], semsc[k]).wait()
            pltpu.make_async_copy(
                rc.at[k], acc.at[scs.at[k]], semsc[k]).wait()
        plsc.subcore_barrier()
        pltpu.sync_copy(acc.at[pl.ds(rbase, STRIPE)],
                        out.at[pl.ds(coff + rbase, STRIPE)])
        if not edge_split:
            pltpu.sync_copy(
                acc.at[pl.ds(rbase, STRIPE)],
                full_out.at[pl.ds(rbase, STRIPE), pl.ds(c * H, H)])

        @pl.when(s == NS - 1)
        def _():
            pltpu.sync_copy(acc.at[pl.ds(NS * STRIPE, TAIL)],
                            out.at[pl.ds(coff + NS * STRIPE, TAIL)])
            if not edge_split:
                pltpu.sync_copy(
                    acc.at[pl.ds(NS * STRIPE, TAIL)],
                    full_out.at[pl.ds(NS * STRIPE, TAIL), pl.ds(c * H, H)])

    return scatter


_scatter_h1 = _make_scatter(H1, edge_split=False)
_scatter_g = _make_scatter(D2, edge_split=True)


@functools.partial(
    pl.kernel,
    out_type=jax.ShapeDtypeStruct((2 * EPAD,), jnp.float32),
    mesh=_mesh,
    scratch_types=[
        pltpu.VMEM((N,), jnp.float32),
        pltpu.VMEM((N,), jnp.float32),
        pltpu.VMEM((CEC,), jnp.int32),
        pltpu.VMEM((CEC,), jnp.int32),
        pltpu.VMEM((CEC,), jnp.float32),
        pltpu.VMEM((CEC,), jnp.float32),
        pltpu.VMEM((CEC,), jnp.float32),
        pltpu.VMEM((CEC,), jnp.float32),
        pltpu.SemaphoreType.DMA,
    ],
    compiler_params=pltpu.CompilerParams(needs_layout_passes=False),
)
def _edge_score(v_p, t_p, rou0, rou1, src0, dst0, s_out,
                r0v, r1v, srcv, dstv, vpv, tpv, sfv, sbv, semi):
    c = lax.axis_index("c")
    s = lax.axis_index("s")
    wid = s * NC + c
    pltpu.sync_copy(rou0, r0v)
    pltpu.sync_copy(rou1, r1v)

    def chunk(jj, _):
        off = (wid + jj * NW) * CEC
        pltpu.async_copy(src0.at[pl.ds(off, CEC)], srcv, semi)
        pltpu.async_copy(dst0.at[pl.ds(off, CEC)], dstv, semi)
        pltpu.async_copy(v_p.at[pl.ds(off, CEC)], vpv, semi)
        pltpu.async_copy(t_p.at[pl.ds(off, CEC)], tpv, semi)
        pltpu.make_async_copy(src0.at[pl.ds(off, CEC)], srcv, semi).wait()
        pltpu.make_async_copy(dst0.at[pl.ds(off, CEC)], dstv, semi).wait()
        pltpu.make_async_copy(v_p.at[pl.ds(off, CEC)], vpv, semi).wait()
        pltpu.make_async_copy(t_p.at[pl.ds(off, CEC)], tpv, semi).wait()
        for g in range(CEC // L):
            sl = pl.ds(g * L, L)
            sv = srcv[sl]
            dv = dstv[sl]
            vp = vpv[sl]
            tp = tpv[sl]
            sfv[sl] = jnp.maximum(vp * plsc.load_gather(r0v, [sv]),
                                  tp * plsc.load_gather(r1v, [sv]))
            sbv[sl] = jnp.maximum(vp * plsc.load_gather(r0v, [dv]),
                                  tp * plsc.load_gather(r1v, [dv]))
        pltpu.sync_copy(sfv, s_out.at[pl.ds(off, CEC)])
        pltpu.sync_copy(sbv, s_out.at[pl.ds(EPAD + off, CEC)])
        return 0

    lax.fori_loop(0, ITC, chunk, 0)


_RB = 1000


def _addn_body(*refs):
    o_ref = refs[-1]
    acc = refs[0][...]
    for r in refs[1:-1]:
        acc = acc + r[...]
    o_ref[...] = acc


def _addn(*xs):
    nrows, ncols = xs[0].shape
    return pl.pallas_call(
        _addn_body,
        grid=(nrows // _RB,),
        in_specs=[pl.BlockSpec((_RB, ncols), lambda i: (i, 0))] * len(xs),
        out_specs=pl.BlockSpec((_RB, ncols), lambda i: (i, 0)),
        out_shape=jax.ShapeDtypeStruct((nrows, ncols), jnp.float32),
    )(*xs)


def _mm_l2_body(f_ref, w_ref, b_ref, o_ref, of_ref):
    y = jnp.dot(f_ref[...], w_ref[...], preferred_element_type=jnp.float32)
    y = y + b_ref[...]
    nrm = jnp.sqrt(jnp.sum(y * y, axis=1, keepdims=True))
    y = y / jnp.maximum(nrm, 1e-12)
    o_ref[0] = y[:, :H1]
    o_ref[1] = y[:, H1:]
    of_ref[...] = y


def _l2_body(x_ref, o_ref, of_ref):
    y = x_ref[...]
    nrm = jnp.sqrt(jnp.sum(y * y, axis=1, keepdims=True))
    y = y / jnp.maximum(nrm, 1e-12)
    o_ref[0] = y[:, :H1]
    o_ref[1] = y[:, H1:]
    of_ref[...] = y


def _mm_l2(feat, W, b):
    nrows = feat.shape[0]
    return pl.pallas_call(
        _mm_l2_body,
        grid=(nrows // _RB,),
        in_specs=[
            pl.BlockSpec((_RB, feat.shape[1]), lambda i: (i, 0)),
            pl.BlockSpec(W.shape, lambda i: (0, 0)),
            pl.BlockSpec((1, D1), lambda i: (0, 0)),
        ],
        out_specs=[pl.BlockSpec((2, _RB, H1), lambda i: (0, i, 0)),
                   pl.BlockSpec((_RB, D1), lambda i: (i, 0))],
        out_shape=[jax.ShapeDtypeStruct((2, nrows, H1), jnp.float32),
                   jax.ShapeDtypeStruct((nrows, D1), jnp.float32)],
    )(feat, W, b.reshape(1, D1))


def _l2split(x):
    nrows = x.shape[0]
    return pl.pallas_call(
        _l2_body,
        grid=(nrows // _RB,),
        in_specs=[pl.BlockSpec((_RB, D1), lambda i: (i, 0))],
        out_specs=[pl.BlockSpec((2, _RB, H1), lambda i: (0, i, 0)),
                   pl.BlockSpec((_RB, D1), lambda i: (i, 0))],
        out_shape=[jax.ShapeDtypeStruct((2, nrows, H1), jnp.float32),
                   jax.ShapeDtypeStruct((nrows, D1), jnp.float32)],
    )(x)


def _prep(feat, W, b, pref):
    nf, n_full = _mm_l2(feat, W, b)
    pf, p_full = _l2split(pref)
    xsplit = jnp.concatenate([pf[0], nf[0], pf[1], nf[1]], axis=0)
    xfull = jnp.concatenate([p_full, n_full], axis=0)
    return xsplit, xfull


def _unsplit(xflat, H):
    return xflat.reshape(2, N, H).transpose(1, 0, 2).reshape(N, 2 * H)


def _zerotail(p):
    return jnp.concatenate([p[:E], jnp.zeros((EPAD - E,), jnp.float32)])


def kernel(edge_index, v_f, t_f, preference_v, W_v, b_v,
           preference_t, W_t, b_t, id_embedding, rou):
    pad = jnp.zeros((EPAD - E,), jnp.int32)
    src0 = jnp.concatenate([edge_index[0], pad])
    dst0 = jnp.concatenate([edge_index[1], pad])

    xv, xvfull = _prep(v_f, W_v, b_v, preference_v)
    xt, xtfull = _prep(t_f, W_t, b_t, preference_t)

    zeros1 = jnp.zeros((2 * N, H1), jnp.float32)
    p1v = _zerotail(_edge_conf(xvfull, src0, dst0))
    x1v, x1vfull = _scatter_h1(xv, src0, dst0, p1v, p1v, zeros1)
    p2v = _zerotail(_edge_conf(x1vfull, src0, dst0))
    x2v, _ = _scatter_h1(x1v, src0, dst0, p2v, p2v, zeros1)

    p1t = _zerotail(_edge_conf(xtfull, src0, dst0))
    x1t, x1tfull = _scatter_h1(xt, src0, dst0, p1t, p1t, zeros1)
    p2t = _zerotail(_edge_conf(x1tfull, src0, dst0))
    x2t, _ = _scatter_h1(x1t, src0, dst0, p2t, p2t, zeros1)

    rou0 = rou[:, 0]
    rou1 = rou[:, 1]
    sflat = _edge_score(p2v, p2t, rou0, rou1, src0, dst0)
    wf = sflat[:EPAD]
    wb = sflat[EPAD:]

    zeros2 = jnp.zeros((2 * N, D2), jnp.float32)
    part1 = _scatter_g(id_embedding, src0, dst0, wf, wb, zeros2)
    x_g = _addn(part1[:N], part1[N:])
    part2 = _scatter_g(x_g, src0, dst0, wf, wb, zeros2)
    id_embed = _addn(x_g, part2[:N], part2[N:])

    vfull = _unsplit(x2v, H1)
    tfull = _unsplit(x2t, H1)
    return jnp.concatenate([id_embed, vfull, tfull], axis=1)


# speedup vs baseline: 1.0807x; 1.0807x over previous
import functools

import jax
import jax.numpy as jnp
from jax import lax
from jax.experimental import pallas as pl
from jax.experimental.pallas import tpu as pltpu
from jax.experimental.pallas import tpu_sc as plsc

NUM_USER = 5000
NUM_ITEM = 5000
N = NUM_USER + NUM_ITEM
E = 160000
EPAD = 163840
D1 = 256
H1 = 128
D2 = 128

NC = 2
NS = 16
L = 16
NW = NC * NS

CEC = 80
CES = 80
ITC = EPAD // CEC // NW
STRIPE = 624
TAIL = N - NS * STRIPE

_mesh = plsc.VectorSubcoreMesh(core_axis_name="c", subcore_axis_name="s")


def _sigmoid(x):
    return 1.0 / (1.0 + jnp.exp(-x))


CCF = 64
ITF = EPAD // CCF // NW


@functools.partial(
    pl.kernel,
    out_type=jax.ShapeDtypeStruct((EPAD,), jnp.float32),
    mesh=_mesh,
    scratch_types=[
        pltpu.VMEM((2, CCF), jnp.int32),
        pltpu.VMEM((2, CCF), jnp.int32),
        pltpu.VMEM((2, CCF, H1), jnp.int32),
        pltpu.VMEM((2, CCF, H1), jnp.int32),
        pltpu.VMEM((CCF, L), jnp.float32),
        pltpu.VMEM((2, CCF), jnp.float32),
        pltpu.SemaphoreType.DMA,
        pltpu.SemaphoreType.DMA,
        pltpu.SemaphoreType.DMA,
        pltpu.SemaphoreType.DMA,
        pltpu.SemaphoreType.DMA,
        pltpu.SemaphoreType.DMA,
    ],
    compiler_params=pltpu.CompilerParams(needs_layout_passes=False),
)
def _edge_conf(xfull, src0, dst0, p_out, srcv, dstv, ra, rc, pacc, pbuf,
               semi0, semi1, semr0, semr1, semp0, semp1):
    semi = (semi0, semi1)
    semr = (semr0, semr1)
    semp = (semp0, semp1)
    c = lax.axis_index("c")
    s = lax.axis_index("s")
    wid = s * NC + c

    def off_of(jj):
        return (wid + jj * NW) * CCF

    def issue_idx(k, jj):
        off = off_of(jj)
        pltpu.async_copy(src0.at[pl.ds(off, CCF)], srcv.at[k], semi[k])
        pltpu.async_copy(dst0.at[pl.ds(off, CCF)], dstv.at[k], semi[k])

    def wait_idx(k, jj):
        off = off_of(jj)
        pltpu.make_async_copy(
            src0.at[pl.ds(off, CCF)], srcv.at[k], semi[k]).wait()
        pltpu.make_async_copy(
            dst0.at[pl.ds(off, CCF)], dstv.at[k], semi[k]).wait()

    def issue_rows(k):
        pltpu.async_copy(xfull.at[srcv.at[k]], ra.at[k], semr[k])
        pltpu.async_copy(xfull.at[dstv.at[k]], rc.at[k], semr[k])

    def wait_rows(k):
        pltpu.make_async_copy(
            xfull.at[srcv.at[k]], ra.at[k], semr[k]).wait()
        pltpu.make_async_copy(
            xfull.at[dstv.at[k]], rc.at[k], semr[k]).wait()

    issue_idx(0, 0)
    wait_idx(0, 0)
    issue_rows(0)
    issue_idx(1, 1)

    def outer(jjj, _):
        for k in (0, 1):
            jj = 2 * jjj + k
            off = off_of(jj)
            wait_rows(k)

            @pl.when(jj + 1 < ITF)
            def _():
                wait_idx(1 - k, jj + 1)
                issue_rows(1 - k)

            @pl.when(jj + 2 < ITF)
            def _():
                issue_idx(k, jj + 2)

            @pl.when(jj >= 2)
            def _():
                poff = off_of(jj - 2)
                pltpu.make_async_copy(
                    pbuf.at[k], p_out.at[pl.ds(poff, CCF)], semp[k]).wait()

            @plsc.parallel_loop(0, CCF, unroll=4)
            def _(i):
                acc = jnp.zeros((L,), jnp.float32)
                acc2 = jnp.zeros((L,), jnp.float32)
                for q in range(H1 // L):
                    sl = pl.ds(q * L, L)
                    wa = ra[k, i, sl]
                    wb = rc[k, i, sl]
                    a_hi = plsc.bitcast(wa & jnp.int32(-65536), jnp.float32)
                    a_lo = plsc.bitcast(wa << 16, jnp.float32)
                    b_hi = plsc.bitcast(wb & jnp.int32(-65536), jnp.float32)
                    b_lo = plsc.bitcast(wb << 16, jnp.float32)
                    acc = acc + a_hi * b_hi
                    acc2 = acc2 + a_lo * b_lo
                pacc[i] = acc + acc2
            for g in range(CCF // L):
                lanes = g * L + lax.iota(jnp.int32, L)
                tot = plsc.load_gather(
                    pacc, [lanes, jnp.zeros((L,), jnp.int32)])
                for q in range(1, L):
                    tot = tot + plsc.load_gather(
                        pacc, [lanes, jnp.full((L,), q, jnp.int32)])
                pbuf[k, pl.ds(g * L, L)] = _sigmoid(tot)
            pltpu.async_copy(pbuf.at[k], p_out.at[pl.ds(off, CCF)], semp[k])
        return 0

    lax.fori_loop(0, ITF // 2, outer, 0)
    for k in (0, 1):
        poff = off_of(ITF - 2 + k)
        pltpu.make_async_copy(
            pbuf.at[k], p_out.at[pl.ds(poff, CCF)], semp[k]).wait()


def _make_scatter(H, edge_split):
    scratch = [
        pltpu.VMEM_SHARED((N, H), jnp.float32),
        pltpu.VMEM((2, CES), jnp.int32),
        pltpu.VMEM((2, CES), jnp.int32),
        pltpu.VMEM((2, CES, H), jnp.float32),
        pltpu.VMEM((2, CES, H), jnp.float32),
        pltpu.VMEM((2, CES), jnp.float32),
        pltpu.VMEM((2, CES), jnp.float32),
        pltpu.VMEM((2, CES), jnp.int32),
        pltpu.VMEM((2, CES), jnp.int32),
        pltpu.SemaphoreType.DMA,
        pltpu.SemaphoreType.DMA,
        pltpu.SemaphoreType.DMA,
        pltpu.SemaphoreType.DMA,
        pltpu.SemaphoreType.DMA,
        pltpu.SemaphoreType.DMA,
    ]
    if not edge_split:
        scratch += [
            pltpu.VMEM((2, CES), jnp.int32),
            pltpu.VMEM((2, CES), jnp.int32),
        ]

    @functools.partial(
        pl.kernel,
        out_type=jax.ShapeDtypeStruct((2 * N, H), jnp.float32),
        mesh=_mesh,
        scratch_types=scratch,
        compiler_params=pltpu.CompilerParams(needs_layout_passes=False),
    )
    def scatter(xtab, src0, dst0, wf, wb, init, out,
                acc, srcv, dstv, ra, rc, wfv, wbv, scs, scd,
                semi0, semi1, semr0, semr1, semsc0, semsc1, *offv):
        semi = (semi0, semi1)
        semr = (semr0, semr1)
        semsc = (semsc0, semsc1)
        c = lax.axis_index("c")
        s = lax.axis_index("s")
        coff = c * N
        rbase = s * STRIPE
        pltpu.sync_copy(init.at[pl.ds(coff + rbase, STRIPE)],
                        acc.at[pl.ds(rbase, STRIPE)])

        @pl.when(s == NS - 1)
        def _():
            pltpu.sync_copy(init.at[pl.ds(coff + NS * STRIPE, TAIL)],
                            acc.at[pl.ds(NS * STRIPE, TAIL)])

        if edge_split:
            wid = c * NS + s
            stride = NW
        else:
            wid = s
            stride = NS
        iters = EPAD // CES // stride

        def off_of(jj):
            return (wid + jj * stride) * CES

        def issue_idx(k, jj):
            off = off_of(jj)
            pltpu.async_copy(src0.at[pl.ds(off, CES)], srcv.at[k], semi[k])
            pltpu.async_copy(dst0.at[pl.ds(off, CES)], dstv.at[k], semi[k])
            pltpu.async_copy(wf.at[pl.ds(off, CES)], wfv.at[k], semi[k])
            pltpu.async_copy(wb.at[pl.ds(off, CES)], wbv.at[k], semi[k])

        def wait_idx(k, jj):
            off = off_of(jj)
            pltpu.make_async_copy(
                src0.at[pl.ds(off, CES)], srcv.at[k], semi[k]).wait()
            pltpu.make_async_copy(
                dst0.at[pl.ds(off, CES)], dstv.at[k], semi[k]).wait()
            pltpu.make_async_copy(
                wf.at[pl.ds(off, CES)], wfv.at[k], semi[k]).wait()
            pltpu.make_async_copy(
                wb.at[pl.ds(off, CES)], wbv.at[k], semi[k]).wait()

        def issue_rows(k):
            if edge_split:
                gsrc, gdst = srcv, dstv
            else:
                gsrc, gdst = offv
                for g in range(CES // L):
                    sl = pl.ds(g * L, L)
                    gsrc[k, sl] = srcv[k, sl] + coff
                    gdst[k, sl] = dstv[k, sl] + coff
            pltpu.async_copy(xtab.at[gsrc.at[k]], ra.at[k], semr[k])
            pltpu.async_copy(xtab.at[gdst.at[k]], rc.at[k], semr[k])

        def wait_rows(k):
            gsrc, gdst = (srcv, dstv) if edge_split else offv
            pltpu.make_async_copy(
                xtab.at[gsrc.at[k]], ra.at[k], semr[k]).wait()
            pltpu.make_async_copy(
                xtab.at[gdst.at[k]], rc.at[k], semr[k]).wait()

        def wait_scatter(k):
            pltpu.make_async_copy(
                ra.at[k], acc.at[scd.at[k]], semsc[k]).wait()
            pltpu.make_async_copy(
                rc.at[k], acc.at[scs.at[k]], semsc[k]).wait()

        issue_idx(0, 0)
        wait_idx(0, 0)
        issue_rows(0)
        issue_idx(1, 1)
        plsc.subcore_barrier()

        def outer(jjj, _):
            for k in (0, 1):
                jj = 2 * jjj + k
                wait_rows(k)

                @pl.when(jj + 1 < iters)
                def _():
                    wait_idx(1 - k, jj + 1)

                    @pl.when(jj >= 1)
                    def _():
                        wait_scatter(1 - k)

                    issue_rows(1 - k)

                for g in range(CES // L):
                    sl = pl.ds(g * L, L)
                    scs[k, sl] = srcv[k, sl]
                    scd[k, sl] = dstv[k, sl]

                @plsc.parallel_loop(0, CES, unroll=4)
                def _(i):
                    wfb = plsc.load_gather(
                        wfv.at[k], [jnp.full((L,), i, jnp.int32)])
                    wbb = plsc.load_gather(
                        wbv.at[k], [jnp.full((L,), i, jnp.int32)])
                    for q in range(H // L):
                        sl = pl.ds(q * L, L)
                        ra[k, i, sl] = ra[k, i, sl] * wfb
                        rc[k, i, sl] = rc[k, i, sl] * wbb
                pltpu.async_copy(ra.at[k], acc.at[scd.at[k]], semsc[k],
                                 add=True)
                pltpu.async_copy(rc.at[k], acc.at[scs.at[k]], semsc[k],
                                 add=True)

                @pl.when(jj + 2 < iters)
                def _():
                    issue_idx(k, jj + 2)
            return 0

        lax.fori_loop(0, iters // 2, outer, 0)
        for k in (0, 1):
            pltpu.make_async_copy(
                ra.at[k], acc.at[scd.at[k]], semsc[k]).wait()
            pltpu.make_async_copy(
                rc.at[k], acc.at[scs.at[k]], semsc[k]).wait()
        plsc.subcore_barrier()
        pltpu.sync_copy(acc.at[pl.ds(rbase, STRIPE)],
                        out.at[pl.ds(coff + rbase, STRIPE)])

        @pl.when(s == NS - 1)
        def _():
            pltpu.sync_copy(acc.at[pl.ds(NS * STRIPE, TAIL)],
                            out.at[pl.ds(coff + NS * STRIPE, TAIL)])

    return scatter


_scatter_h1 = _make_scatter(H1, edge_split=False)
_scatter_g = _make_scatter(D2, edge_split=True)


@functools.partial(
    pl.kernel,
    out_type=jax.ShapeDtypeStruct((2 * EPAD,), jnp.float32),
    mesh=_mesh,
    scratch_types=[
        pltpu.VMEM((N,), jnp.float32),
        pltpu.VMEM((N,), jnp.float32),
        pltpu.VMEM((CEC,), jnp.int32),
        pltpu.VMEM((CEC,), jnp.int32),
        pltpu.VMEM((CEC,), jnp.float32),
        pltpu.VMEM((CEC,), jnp.float32),
        pltpu.VMEM((CEC,), jnp.float32),
        pltpu.VMEM((CEC,), jnp.float32),
        pltpu.SemaphoreType.DMA,
    ],
    compiler_params=pltpu.CompilerParams(needs_layout_passes=False),
)
def _edge_score(v_p, t_p, rou0, rou1, src0, dst0, s_out,
                r0v, r1v, srcv, dstv, vpv, tpv, sfv, sbv, semi):
    c = lax.axis_index("c")
    s = lax.axis_index("s")
    wid = s * NC + c
    pltpu.sync_copy(rou0, r0v)
    pltpu.sync_copy(rou1, r1v)

    def chunk(jj, _):
        off = (wid + jj * NW) * CEC
        pltpu.async_copy(src0.at[pl.ds(off, CEC)], srcv, semi)
        pltpu.async_copy(dst0.at[pl.ds(off, CEC)], dstv, semi)
        pltpu.async_copy(v_p.at[pl.ds(off, CEC)], vpv, semi)
        pltpu.async_copy(t_p.at[pl.ds(off, CEC)], tpv, semi)
        pltpu.make_async_copy(src0.at[pl.ds(off, CEC)], srcv, semi).wait()
        pltpu.make_async_copy(dst0.at[pl.ds(off, CEC)], dstv, semi).wait()
        pltpu.make_async_copy(v_p.at[pl.ds(off, CEC)], vpv, semi).wait()
        pltpu.make_async_copy(t_p.at[pl.ds(off, CEC)], tpv, semi).wait()
        for g in range(CEC // L):
            sl = pl.ds(g * L, L)
            sv = srcv[sl]
            dv = dstv[sl]
            vp = vpv[sl]
            tp = tpv[sl]
            sfv[sl] = jnp.maximum(vp * plsc.load_gather(r0v, [sv]),
                                  tp * plsc.load_gather(r1v, [sv]))
            sbv[sl] = jnp.maximum(vp * plsc.load_gather(r0v, [dv]),
                                  tp * plsc.load_gather(r1v, [dv]))
        pltpu.sync_copy(sfv, s_out.at[pl.ds(off, CEC)])
        pltpu.sync_copy(sbv, s_out.at[pl.ds(EPAD + off, CEC)])
        return 0

    lax.fori_loop(0, ITC, chunk, 0)


_RB = 1000


def _addn_body(*refs):
    o_ref = refs[-1]
    acc = refs[0][...]
    for r in refs[1:-1]:
        acc = acc + r[...]
    o_ref[...] = acc


def _addn(*xs):
    nrows, ncols = xs[0].shape
    return pl.pallas_call(
        _addn_body,
        grid=(nrows // _RB,),
        in_specs=[pl.BlockSpec((_RB, ncols), lambda i: (i, 0))] * len(xs),
        out_specs=pl.BlockSpec((_RB, ncols), lambda i: (i, 0)),
        out_shape=jax.ShapeDtypeStruct((nrows, ncols), jnp.float32),
    )(*xs)


def _pack_bf16_words(y):
    lo = jax.lax.bitcast_convert_type(
        y[:, :H1].astype(jnp.bfloat16), jnp.uint16).astype(jnp.int32)
    hi = jax.lax.bitcast_convert_type(
        y[:, H1:].astype(jnp.bfloat16), jnp.uint16).astype(jnp.int32)
    return lo | (hi << 16)


def _mm_l2_body(f_ref, w_ref, b_ref, o_ref, of_ref):
    y = jnp.dot(f_ref[...], w_ref[...], preferred_element_type=jnp.float32)
    y = y + b_ref[...]
    nrm = jnp.sqrt(jnp.sum(y * y, axis=1, keepdims=True))
    y = y / jnp.maximum(nrm, 1e-12)
    o_ref[0] = y[:, :H1]
    o_ref[1] = y[:, H1:]
    of_ref[...] = _pack_bf16_words(y)


def _l2_body(x_ref, o_ref, of_ref):
    y = x_ref[...]
    nrm = jnp.sqrt(jnp.sum(y * y, axis=1, keepdims=True))
    y = y / jnp.maximum(nrm, 1e-12)
    o_ref[0] = y[:, :H1]
    o_ref[1] = y[:, H1:]
    of_ref[...] = _pack_bf16_words(y)


def _mm_l2(feat, W, b):
    nrows = feat.shape[0]
    return pl.pallas_call(
        _mm_l2_body,
        grid=(nrows // _RB,),
        in_specs=[
            pl.BlockSpec((_RB, feat.shape[1]), lambda i: (i, 0)),
            pl.BlockSpec(W.shape, lambda i: (0, 0)),
            pl.BlockSpec((1, D1), lambda i: (0, 0)),
        ],
        out_specs=[pl.BlockSpec((2, _RB, H1), lambda i: (0, i, 0)),
                   pl.BlockSpec((_RB, H1), lambda i: (i, 0))],
        out_shape=[jax.ShapeDtypeStruct((2, nrows, H1), jnp.float32),
                   jax.ShapeDtypeStruct((nrows, H1), jnp.int32)],
    )(feat, W, b.reshape(1, D1))


def _l2split(x):
    nrows = x.shape[0]
    return pl.pallas_call(
        _l2_body,
        grid=(nrows // _RB,),
        in_specs=[pl.BlockSpec((_RB, D1), lambda i: (i, 0))],
        out_specs=[pl.BlockSpec((2, _RB, H1), lambda i: (0, i, 0)),
                   pl.BlockSpec((_RB, H1), lambda i: (i, 0))],
        out_shape=[jax.ShapeDtypeStruct((2, nrows, H1), jnp.float32),
                   jax.ShapeDtypeStruct((nrows, H1), jnp.int32)],
    )(x)


def _cat_bf16_body(a_ref, b_ref, o_ref):
    y = jnp.concatenate([a_ref[...], b_ref[...]], axis=1)
    o_ref[...] = _pack_bf16_words(y)


def _cat_bf16(xsplit):
    return pl.pallas_call(
        _cat_bf16_body,
        grid=(N // _RB,),
        in_specs=[pl.BlockSpec((_RB, H1), lambda i: (i, 0))] * 2,
        out_specs=pl.BlockSpec((_RB, H1), lambda i: (i, 0)),
        out_shape=jax.ShapeDtypeStruct((N, H1), jnp.int32),
    )(xsplit[:N], xsplit[N:])


def _prep(feat, W, b, pref):
    nf, n_full = _mm_l2(feat, W, b)
    pf, p_full = _l2split(pref)
    xsplit = jnp.concatenate([pf[0], nf[0], pf[1], nf[1]], axis=0)
    xfull = jnp.concatenate([p_full, n_full], axis=0)
    return xsplit, xfull


def _unsplit(xflat, H):
    return xflat.reshape(2, N, H).transpose(1, 0, 2).reshape(N, 2 * H)


def _zerotail(p):
    return jnp.concatenate([p[:E], jnp.zeros((EPAD - E,), jnp.float32)])


def kernel(edge_index, v_f, t_f, preference_v, W_v, b_v,
           preference_t, W_t, b_t, id_embedding, rou):
    pad = jnp.zeros((EPAD - E,), jnp.int32)
    src0 = jnp.concatenate([edge_index[0], pad])
    dst0 = jnp.concatenate([edge_index[1], pad])

    xv, xvfull = _prep(v_f, W_v, b_v, preference_v)
    xt, xtfull = _prep(t_f, W_t, b_t, preference_t)

    zeros1 = jnp.zeros((2 * N, H1), jnp.float32)
    p1v = _zerotail(_edge_conf(xvfull, src0, dst0))
    x1v = _scatter_h1(xv, src0, dst0, p1v, p1v, zeros1)
    p2v = _zerotail(_edge_conf(_cat_bf16(x1v), src0, dst0))
    x2v = _scatter_h1(x1v, src0, dst0, p2v, p2v, zeros1)

    p1t = _zerotail(_edge_conf(xtfull, src0, dst0))
    x1t = _scatter_h1(xt, src0, dst0, p1t, p1t, zeros1)
    p2t = _zerotail(_edge_conf(_cat_bf16(x1t), src0, dst0))
    x2t = _scatter_h1(x1t, src0, dst0, p2t, p2t, zeros1)

    rou0 = rou[:, 0]
    rou1 = rou[:, 1]
    sflat = _edge_score(p2v, p2t, rou0, rou1, src0, dst0)
    wf = sflat[:EPAD]
    wb = sflat[EPAD:]

    zeros2 = jnp.zeros((2 * N, D2), jnp.float32)
    part1 = _scatter_g(id_embedding, src0, dst0, wf, wb, zeros2)
    x_g = _addn(part1[:N], part1[N:])
    part2 = _scatter_g(x_g, src0, dst0, wf, wb, zeros2)
    id_embed = _addn(x_g, part2[:N], part2[N:])

    vfull = _unsplit(x2v, H1)
    tfull = _unsplit(x2t, H1)
    return jnp.concatenate([id_embed, vfull, tfull], axis=1)
